# baseline (speedup 1.0000x reference)
import functools

import jax
import jax.numpy as jnp
from jax import lax
from jax.experimental import pallas as pl
from jax.experimental.pallas import tpu as pltpu
from jax.experimental.pallas import tpu_sc as plsc

SEQ = 200
EXP = 32
BG = 128
NW = 32
INFLIGHT = 8


def _rank_body(ts_ref, g_ref):
    tst = ts_ref[...].T
    jio = lax.broadcasted_iota(jnp.int32, (SEQ, BG), 0)
    acc = jnp.zeros((SEQ, BG), jnp.float32)
    for k in range(SEQ):
        bk = jnp.broadcast_to(tst[k:k + 1, :], (SEQ, BG))
        lt = bk < tst
        tie = (bk == tst) & (jio > k)
        acc = acc + jnp.where(lt | tie, 1.0, 0.0)
    b = pl.program_id(0) * BG + lax.broadcasted_iota(jnp.int32, (BG, SEQ), 0)
    g_ref[...] = acc.T.astype(jnp.int32) + b * SEQ


def _ranks_tc(ts):
    batch = ts.shape[0]
    ngrp = batch // BG
    return pl.pallas_call(
        _rank_body,
        grid=(ngrp,),
        in_specs=[pl.BlockSpec((BG, SEQ), lambda i: (i, 0))],
        out_specs=pl.BlockSpec((BG, SEQ), lambda i: (i, 0)),
        out_shape=jax.ShapeDtypeStruct((batch, SEQ), jnp.int32),
    )(ts)


def _scatter_sc(g, embedding, batch):
    flat = batch * SEQ
    ngrp = batch // BG
    gpw = ngrp // NW
    mesh = plsc.VectorSubcoreMesh(core_axis_name="c", subcore_axis_name="s")

    @functools.partial(
        pl.kernel,
        mesh=mesh,
        out_type=jax.ShapeDtypeStruct((flat, EXP), jnp.float32),
        scratch_types=[
            pltpu.VMEM((BG, SEQ), jnp.int32),
            pltpu.VMEM((BG, SEQ), jnp.int32),
            pltpu.VMEM((SEQ, EXP), jnp.float32),
            pltpu.SemaphoreType.DMA,
            pltpu.SemaphoreType.DMA,
        ],
        compiler_params=pltpu.CompilerParams(use_tc_tiling_on_sc=False),
    )
    def k(g_hbm, emb_hbm, out_hbm, ids0_v, ids1_v, tbl_v, sem, lsem):
        wid = lax.axis_index("s") * 2 + lax.axis_index("c")
        pltpu.sync_copy(emb_hbm, tbl_v)

        slots = (ids0_v, ids1_v)

        def load(t, slot):
            base = (wid * gpw + t) * BG
            return pltpu.make_async_copy(
                g_hbm.at[pl.ds(base, BG)], slots[slot], lsem)

        def chunk_copy(ids_v, c):
            return pltpu.make_async_copy(
                tbl_v, out_hbm.at[ids_v.at[c]], sem)

        load(0, 0).start()
        for t in range(gpw):
            ids_v = slots[t & 1]
            load(t, t & 1).wait()
            if t + 1 < gpw:
                load(t + 1, (t + 1) & 1).start()

            def scat_step(c, carry):
                chunk_copy(ids_v, c).start()

                @pl.when(c >= INFLIGHT)
                def _():
                    chunk_copy(ids_v, c - INFLIGHT).wait()
                return carry

            lax.fori_loop(0, BG, scat_step, 0)
            for c in range(BG - INFLIGHT, BG):
                chunk_copy(ids_v, c).wait()

    return k(g, embedding)


def kernel(ts, embedding):
    batch = ts.shape[0]
    g = _ranks_tc(ts)
    out = _scatter_sc(g, embedding, batch)
    return out.reshape(batch, SEQ, EXP)
